# baseline (speedup 1.0000x reference)
import jax
import jax.numpy as jnp
from jax import lax
from jax.experimental import pallas as pl
from jax.experimental.pallas import tpu as pltpu
from jax.experimental.pallas import tpu_sc as plsc

NUM_CORES = 2
NUM_SUBCORES = 16
LANES = 16
NW = NUM_CORES * NUM_SUBCORES

BATCH = 16384
EMBED_DIM = 32
PACK = 4
PROW = PACK * EMBED_DIM
B_PER_W = BATCH // NW
CHUNK = 128
N_CHUNK = B_PER_W // CHUNK
BLK_PER_CHUNK = CHUNK // LANES
W_PAD = 48


def _gmf_body(uid_hbm, iid_hbm, ut_hbm, it_hbm, w_hbm, b_hbm, out_hbm,
              idxu_v, idxi_v, rowu_v, rowi_v, ubuf, ibuf, w_v, b_v, out_v,
              sems):
    wid = lax.axis_index("c") * NUM_SUBCORES + lax.axis_index("s")
    crow = wid * N_CHUNK

    pltpu.sync_copy(uid_hbm.at[pl.ds(crow, N_CHUNK)], idxu_v)
    pltpu.sync_copy(iid_hbm.at[pl.ds(crow, N_CHUNK)], idxi_v)
    pltpu.sync_copy(w_hbm, w_v)
    pltpu.sync_copy(b_hbm, b_v)

    for c in range(N_CHUNK):
        for k in range(BLK_PER_CHUNK):
            s = pl.ds(k * LANES, LANES)
            rowu_v[c, s] = lax.shift_right_logical(idxu_v[c, s], 2)
            rowi_v[c, s] = lax.shift_right_logical(idxi_v[c, s], 2)

    def fire(c):
        buf = c % 2
        return (pltpu.async_copy(ut_hbm.at[rowu_v.at[c]], ubuf.at[buf],
                                 sems.at[buf]),
                pltpu.async_copy(it_hbm.at[rowi_v.at[c]], ibuf.at[buf],
                                 sems.at[buf]))

    w_splats = [
        plsc.load_gather(w_v, [jnp.full((LANES,), d + 1, jnp.int32)])
        for d in range(EMBED_DIM)
    ]
    bias = b_v[...]
    lane_iota = lax.broadcasted_iota(jnp.int32, (LANES,), 0)

    pending = fire(0)
    for c in range(N_CHUNK):
        nxt = fire(c + 1) if c + 1 < N_CHUNK else None
        pending[0].wait()
        pending[1].wait()
        buf = c % 2
        for k in range(BLK_PER_CHUNK):
            s = pl.ds(k * LANES, LANES)
            sub_u = lax.shift_left(jnp.bitwise_and(idxu_v[c, s], 3), 5)
            sub_i = lax.shift_left(jnp.bitwise_and(idxi_v[c, s], 3), 5)
            b_loc = k * LANES + lane_iota
            acc = bias
            for d in range(EMBED_DIM):
                ug = plsc.load_gather(ubuf.at[buf], [b_loc, sub_u + d])
                ig = plsc.load_gather(ibuf.at[buf], [b_loc, sub_i + d])
                acc = acc + (ug * ig) * w_splats[d]
            out_v[pl.ds(c * CHUNK + k * LANES, LANES)] = acc
        pending = nxt

    pltpu.sync_copy(out_v, out_hbm.at[pl.ds(wid * B_PER_W, B_PER_W)])


@jax.jit
def _gmf(user_ids, item_ids, user_table, item_table, fc_w_pad, fc_b16):
    mesh = plsc.VectorSubcoreMesh(
        core_axis_name="c", subcore_axis_name="s",
        num_cores=NUM_CORES, num_subcores=NUM_SUBCORES)
    f = pl.kernel(
        _gmf_body,
        out_type=jax.ShapeDtypeStruct((BATCH,), jnp.float32),
        mesh=mesh,
        compiler_params=pltpu.CompilerParams(needs_layout_passes=False),
        scratch_types=[
            pltpu.VMEM((N_CHUNK, CHUNK), jnp.int32),
            pltpu.VMEM((N_CHUNK, CHUNK), jnp.int32),
            pltpu.VMEM((N_CHUNK, CHUNK), jnp.int32),
            pltpu.VMEM((N_CHUNK, CHUNK), jnp.int32),
            pltpu.VMEM((2, CHUNK, PROW), jnp.float32),
            pltpu.VMEM((2, CHUNK, PROW), jnp.float32),
            pltpu.VMEM((W_PAD,), jnp.float32),
            pltpu.VMEM((LANES,), jnp.float32),
            pltpu.VMEM((B_PER_W,), jnp.float32),
            pltpu.SemaphoreType.DMA((2,)),
        ],
    )
    return f(user_ids.reshape(BATCH // CHUNK, CHUNK),
             item_ids.reshape(BATCH // CHUNK, CHUNK),
             user_table.reshape(-1, PROW), item_table.reshape(-1, PROW),
             fc_w_pad, fc_b16)


def kernel(user_ids, item_ids, user_table, item_table, fc_w, fc_b):
    w = fc_w.reshape(EMBED_DIM)
    fc_w_pad = jnp.zeros((W_PAD,), jnp.float32).at[1:EMBED_DIM + 1].set(w)
    fc_b16 = jnp.broadcast_to(fc_b, (LANES,))
    return _gmf(user_ids.astype(jnp.int32), item_ids.astype(jnp.int32),
                user_table, item_table, fc_w_pad, fc_b16)
